# baseline (speedup 1.0000x reference)
import functools

import jax
import jax.numpy as jnp
from jax.experimental import pallas as pl
from jax.experimental.pallas import tpu as pltpu

N = 10000
K = 32
H = 128
V = 21
M = 256
RBF = 16
L = 3

R = 400
NBLK = N // R
NP = 10048


def _knn_body(xr_ref, xpt_ref, nbr_ref, e_ref, d2_ref):
    f32 = jnp.float32
    xr = xr_ref[...]
    xpt = xpt_ref[...]
    sq_r = jnp.sum(xr * xr, axis=1, keepdims=True)
    sq_c = jnp.sum(xpt * xpt, axis=0, keepdims=True)
    dots = jnp.dot(xr, xpt, preferred_element_type=f32)
    iota = jax.lax.broadcasted_iota(jnp.int32, (R, NP), 1)
    d2 = sq_r + sq_c - 2.0 * dots
    d2_ref[...] = jnp.where(iota < N, d2, jnp.inf)

    idxs = []
    vals = []
    for _ in range(K):
        v = d2_ref[...]
        val = jnp.min(v, axis=1, keepdims=True)
        idx = jnp.min(jnp.where(v == val, iota, NP), axis=1, keepdims=True)
        idxs.append(idx)
        vals.append(val)
        d2_ref[...] = jnp.where(iota == idx, jnp.inf, v)

    nbr_ref[...] = jnp.concatenate(idxs, axis=1)
    dmat = jnp.concatenate(vals, axis=1)
    d = jnp.sqrt(jnp.maximum(dmat, 0.0) + 1e-6)
    centers = jax.lax.broadcasted_iota(
        jnp.int32, (1, 1, RBF), 2).astype(f32) * (20.0 / (RBF - 1))
    sigma = 20.0 / RBF
    e3 = jnp.exp(-(((d[:, :, None] - centers) / sigma) ** 2))
    e_ref[...] = e3.reshape(R * K, RBF)


_knn_call = pl.pallas_call(
    _knn_body,
    grid=(NBLK,),
    in_specs=[
        pl.BlockSpec((R, 3), lambda i: (i, 0)),
        pl.BlockSpec((3, NP), lambda i: (0, 0)),
    ],
    out_specs=[
        pl.BlockSpec((R, K), lambda i: (i, 0)),
        pl.BlockSpec((R * K, RBF), lambda i: (i, 0)),
    ],
    out_shape=[
        jax.ShapeDtypeStruct((N, K), jnp.int32),
        jax.ShapeDtypeStruct((N * K, RBF), jnp.float32),
    ],
    scratch_shapes=[pltpu.VMEM((R, NP), jnp.float32)],
)


def _layer_body(h_ref, hj_ref, e_ref, w1a_ref, b1_ref, w1c_ref, wm2_ref,
                b2_ref, wu_ref, bu_ref, out_ref):
    f32 = jnp.float32
    h_blk = h_ref[...]
    a = jnp.dot(h_blk, w1a_ref[...], preferred_element_type=f32) + b1_ref[...]
    a_e = jnp.broadcast_to(a[:, None, :], (R, K, H)).reshape(R * K, H)
    ec = jnp.dot(e_ref[...], w1c_ref[...], preferred_element_type=f32)
    m1 = jnp.maximum(a_e + hj_ref[...] + ec, 0.0)
    m2 = jnp.maximum(
        jnp.dot(m1, wm2_ref[...], preferred_element_type=f32) + b2_ref[...],
        0.0)
    agg = jnp.mean(m2.reshape(R, K, H), axis=1)
    u = jnp.maximum(
        jnp.dot(h_blk, wu_ref[0], preferred_element_type=f32)
        + jnp.dot(agg, wu_ref[1], preferred_element_type=f32)
        + bu_ref[...], 0.0)
    out_ref[...] = h_blk + u


def _full(shape):
    return pl.BlockSpec(shape, lambda i: (0,) * len(shape))


_layer_call = pl.pallas_call(
    _layer_body,
    grid=(NBLK,),
    in_specs=[
        pl.BlockSpec((R, H), lambda i: (i, 0)),
        pl.BlockSpec((R * K, H), lambda i: (i, 0)),
        pl.BlockSpec((R * K, RBF), lambda i: (i, 0)),
        _full((H, H)),
        _full((1, H)),
        _full((RBF, H)),
        _full((H, H)),
        _full((1, H)),
        _full((2, H, H)),
        _full((1, H)),
    ],
    out_specs=pl.BlockSpec((R, H), lambda i: (i, 0)),
    out_shape=jax.ShapeDtypeStruct((N, H), jnp.float32),
)


def kernel(X, S, positions, aa_idx, W_s, Wm1, bm1, Wm2, bm2, Wu, bu,
           Wddg, bddg, Wdtm, bdtm):
    xpt = jnp.pad(X, ((0, NP - N), (0, 0))).T
    nbr, e_flat = _knn_call(X, xpt)
    nbr_flat = nbr.reshape(N * K)

    emb = W_s[S]
    h = emb
    for l in range(L):
        w1a = Wm1[l, :H]
        w1b = Wm1[l, H:2 * H]
        w1c = Wm1[l, 2 * H:]
        g = h @ w1b
        hj = g[nbr_flat]
        h = _layer_call(h, hj, e_flat, w1a, bm1[l][None, :], w1c, Wm2[l],
                        bm2[l][None, :], Wu[l].reshape(2, H, H),
                        bu[l][None, :])

    hid = h[positions]
    emb_m = emb[positions]
    lin = jnp.concatenate([hid, emb_m], axis=-1)
    ddg = lin @ Wddg + bddg
    dtm = lin @ Wdtm + bdtm
    ddG = jnp.take_along_axis(ddg, aa_idx[:, None], axis=1)[:, 0]
    dTm = jnp.take_along_axis(dtm, aa_idx[:, None], axis=1)[:, 0]
    return jnp.stack([ddG, dTm], axis=0)
